# Initial kernel scaffold; baseline (speedup 1.0000x reference)
import jax
import jax.numpy as jnp
from jax.experimental import pallas as pl
from jax.experimental.pallas import tpu as pltpu

_B = 64
_D = 128
_H = 64
_K = 16
_M = _D * _D - _D
_TAU = 1.0
_EPS = 1e-10
_NEG = -1e30
_BB = 16
_R = _D + 8


def _reduce2(op, x):
    return op(op(x, axis=2, keepdims=True), axis=1, keepdims=True)


def _body(pos_ref, mage_ref, w_ref, mp_ref, gp_ref, md_ref, gd_ref, dl_ref,
          samp_ref, adj_ref, logp_ref):
    f32 = jnp.float32
    hi = jax.lax.Precision.HIGHEST

    pos = pos_ref[...]
    mage = mage_ref[...]
    w = w_ref[...]
    tmp = jnp.dot(pos.reshape(_BB * _D, _H), w,
                  preferred_element_type=f32, precision=hi).reshape(_BB, _D, _H)
    scores = jax.lax.dot_general(
        tmp, mage, (((2,), (2,)), ((0,), (0,))),
        preferred_element_type=f32, precision=hi)

    mp = mp_ref[...]
    gp = gp_ref[...]
    row = jax.lax.broadcasted_iota(jnp.int32, (_BB, _D, _D), 1)
    col = jax.lax.broadcasted_iota(jnp.int32, (_BB, _D, _D), 2)
    below = col < row
    am = jnp.where(below, mp, pltpu.roll(mp, 1, 2))
    ag = jnp.where(below, gp, pltpu.roll(gp, 1, 2))
    diag = col == row
    mlog = jnp.where(diag, _NEG, am * -1000000.0 + scores * (1.0 - am))
    keys = jnp.where(diag, _NEG, mlog + ag)

    dl = dl_ref[...]
    md = md_ref[...]
    gd = gd_ref[...]
    lane1 = jax.lax.broadcasted_iota(jnp.int32, (_BB, 1, _D), 2)
    mld = jnp.where(lane1 < _K, md * -1000000.0 + dl * (1.0 - md), _NEG)
    kd = jnp.where(lane1 < _K, mld + gd, _NEG)
    sub8 = jax.lax.broadcasted_iota(jnp.int32, (_BB, 8, _D), 1)
    mld8 = jnp.where(sub8 == 0, jnp.broadcast_to(mld, (_BB, 8, _D)), _NEG)
    kd8 = jnp.where(sub8 == 0, jnp.broadcast_to(kd, (_BB, 8, _D)), _NEG)

    logits = jnp.concatenate([mlog, mld8], axis=1)
    keys_f = jnp.concatenate([keys, kd8], axis=1)

    def samp_it(_, carry):
        s, onehot, khot = carry
        s = s + jnp.log(jnp.maximum(1.0 - onehot, _EPS))
        m = _reduce2(jnp.max, s)
        p = jnp.exp((s - m) / _TAU)
        onehot = p / _reduce2(jnp.sum, p)
        return s, onehot, khot + onehot

    zeros = jnp.zeros((_BB, _R, _D), f32)
    _, _, khot = jax.lax.fori_loop(0, _K, samp_it, (keys_f, zeros, zeros))

    rowf = jax.lax.broadcasted_iota(jnp.int32, (_BB, _R, _D), 1)
    colf = jax.lax.broadcasted_iota(jnp.int32, (_BB, _R, _D), 2)
    flat = (rowf * _D + colf).astype(f32)

    def lp_it(_, carry):
        selmask, logp = carry
        mk = keys_f + selmask * _NEG
        m = _reduce2(jnp.max, mk)
        idx = _reduce2(jnp.min, jnp.where(mk == m, flat, 3.4e38))
        sel = flat == idx
        sel_logit = _reduce2(jnp.sum, jnp.where(sel, logits, 0.0))
        ml = logits + selmask * -1e9
        mx = _reduce2(jnp.max, ml)
        lse = jnp.log(_reduce2(jnp.sum, jnp.exp(ml - mx))) + mx
        logp = logp + sel_logit - lse
        return selmask + sel.astype(f32), logp

    _, logp = jax.lax.fori_loop(
        0, _K, lp_it, (zeros, jnp.zeros((_BB, 1, 1), f32)))

    khot_main = khot[:, :_D, :]
    adj_ref[...] = jnp.where(diag, 0.0, khot_main)
    samp_ref[...] = jnp.where(below, khot_main, pltpu.roll(khot_main, -1, 2))
    logp_ref[...] = jnp.broadcast_to(logp.reshape(_BB, 1), (_BB, _D))


_GRID_KWARGS = dict(
    grid=(_B // _BB,),
    in_specs=[
        pl.BlockSpec((_BB, _D, _H), lambda i: (i, 0, 0)),
        pl.BlockSpec((_BB, _D, _H), lambda i: (i, 0, 0)),
        pl.BlockSpec((_H, _H), lambda i: (0, 0)),
        pl.BlockSpec((_BB, _D, _D), lambda i: (i, 0, 0)),
        pl.BlockSpec((_BB, _D, _D), lambda i: (i, 0, 0)),
        pl.BlockSpec((_BB, 1, _D), lambda i: (i, 0, 0)),
        pl.BlockSpec((_BB, 1, _D), lambda i: (i, 0, 0)),
        pl.BlockSpec((1, _D), lambda i: (0, 0)),
    ],
    out_specs=[
        pl.BlockSpec((_BB, _D, _D), lambda i: (i, 0, 0)),
        pl.BlockSpec((_BB, _D, _D), lambda i: (i, 0, 0)),
        pl.BlockSpec((_BB, _D), lambda i: (i, 0)),
    ],
    out_shape=[
        jax.ShapeDtypeStruct((_B, _D, _D), jnp.float32),
        jax.ShapeDtypeStruct((_B, _D, _D), jnp.float32),
        jax.ShapeDtypeStruct((_B, _D), jnp.float32),
    ],
)


@jax.jit
def kernel(pos_embed, mage_embed, mask, gumbel, W_policy, dummy_logits):
    mp = jnp.pad(mask[:, :_M].reshape(_B, _D, _D - 1), ((0, 0), (0, 0), (0, 1)))
    gp = jnp.pad(gumbel[:, :_M].reshape(_B, _D, _D - 1), ((0, 0), (0, 0), (0, 1)))
    md = jnp.pad(mask[:, _M:].reshape(_B, 1, _K), ((0, 0), (0, 0), (0, _D - _K)))
    gd = jnp.pad(gumbel[:, _M:].reshape(_B, 1, _K), ((0, 0), (0, 0), (0, _D - _K)))
    dl = jnp.pad(dummy_logits.reshape(1, _K), ((0, 0), (0, _D - _K)))

    samp_p, adj, logp = pl.pallas_call(_body, **_GRID_KWARGS)(
        pos_embed, mage_embed, W_policy, mp, gp, md, gd, dl)

    sample = samp_p[:, :, :_D - 1].reshape(_B, _M)
    return (sample, adj, logp[:, 0])


# speedup vs baseline: 5.7423x; 5.7423x over previous
import jax
import jax.numpy as jnp
from jax.experimental import pallas as pl
from jax.experimental.pallas import tpu as pltpu

_B = 64
_D = 128
_H = 64
_K = 16
_M = _D * _D - _D
_TAU = 1.0
_EPS = 1e-10
_NEG = -1e30
_BB = 16
_R = _D + 8


def _reduce2(op, x):
    return op(op(x, axis=2, keepdims=True), axis=1, keepdims=True)


def _body(pos_ref, mage_ref, w_ref, mp_ref, gp_ref, md_ref, gd_ref, dl_ref,
          samp_ref, adj_ref, logp_ref):
    f32 = jnp.float32

    pos = pos_ref[...]
    mage = mage_ref[...]
    w = w_ref[...]
    tmp = jnp.dot(pos.reshape(_BB * _D, _H), w,
                  preferred_element_type=f32).reshape(_BB, _D, _H)
    scores = jax.lax.dot_general(
        tmp, mage, (((2,), (2,)), ((0,), (0,))),
        preferred_element_type=f32)

    mp = mp_ref[...]
    gp = gp_ref[...]
    row = jax.lax.broadcasted_iota(jnp.int32, (_BB, _D, _D), 1)
    col = jax.lax.broadcasted_iota(jnp.int32, (_BB, _D, _D), 2)
    below = col < row
    am = jnp.where(below, mp, pltpu.roll(mp, 1, 2))
    ag = jnp.where(below, gp, pltpu.roll(gp, 1, 2))
    diag = col == row
    mlog = jnp.where(diag, _NEG, am * -1000000.0 + scores * (1.0 - am))
    keys = jnp.where(diag, _NEG, mlog + ag)

    dl = dl_ref[...]
    md = md_ref[...]
    gd = gd_ref[...]
    lane1 = jax.lax.broadcasted_iota(jnp.int32, (_BB, 1, _D), 2)
    mld = jnp.where(lane1 < _K, md * -1000000.0 + dl * (1.0 - md), _NEG)
    kd = jnp.where(lane1 < _K, mld + gd, _NEG)
    sub8 = jax.lax.broadcasted_iota(jnp.int32, (_BB, 8, _D), 1)
    mld8 = jnp.where(sub8 == 0, jnp.broadcast_to(mld, (_BB, 8, _D)), _NEG)
    kd8 = jnp.where(sub8 == 0, jnp.broadcast_to(kd, (_BB, 8, _D)), _NEG)

    logits = jnp.concatenate([mlog, mld8], axis=1)
    keys_f = jnp.concatenate([keys, kd8], axis=1)

    rowf = jax.lax.broadcasted_iota(jnp.int32, (_BB, _R, _D), 1)
    colf = jax.lax.broadcasted_iota(jnp.int32, (_BB, _R, _D), 2)
    flat = (rowf * _D + colf).astype(f32)

    def it(i, carry):
        s, onehot, khot, mk, selmask, sel_store = carry
        s = s + jnp.log(jnp.maximum(1.0 - onehot, _EPS))
        m = _reduce2(jnp.max, s)
        p = jnp.exp(s - m)
        onehot = p / _reduce2(jnp.sum, p)
        khot = khot + onehot
        km = _reduce2(jnp.max, mk)
        idx = _reduce2(jnp.min, jnp.where(mk == km, flat, 3.4e38))
        sel = flat == idx
        sel_logit = _reduce2(jnp.sum, jnp.where(sel, logits, 0.0))
        sel_store = jnp.where(lane1 == i, sel_logit, sel_store)
        mk = jnp.where(sel, _NEG, mk)
        selmask = selmask + sel.astype(f32)
        return s, onehot, khot, mk, selmask, sel_store

    zeros = jnp.zeros((_BB, _R, _D), f32)
    _, _, khot, _, selmask, sel_store = jax.lax.fori_loop(
        0, _K, it,
        (keys_f, zeros, zeros, keys_f, zeros, jnp.zeros((_BB, 1, _D), f32)))

    ml = logits + selmask * -1e9
    mxf = _reduce2(jnp.max, ml)
    lse = jnp.log(_reduce2(jnp.sum, jnp.exp(ml - mxf))) + mxf
    logp = jnp.zeros((_BB, 1, 1), f32)
    for i in range(_K - 1, -1, -1):
        sl = jnp.sum(jnp.where(lane1 == i, sel_store, 0.0),
                     axis=2, keepdims=True)
        mab = jnp.maximum(lse, sl)
        lse = mab + jnp.log(jnp.exp(lse - mab) + jnp.exp(sl - mab))
        logp = logp + sl - lse

    khot_main = khot[:, :_D, :]
    adj_ref[...] = jnp.where(diag, 0.0, khot_main)
    samp_ref[...] = jnp.where(below, khot_main, pltpu.roll(khot_main, _D - 1, 2))
    logp_ref[...] = jnp.broadcast_to(logp.reshape(_BB, 1), (_BB, _D))


_GRID_KWARGS = dict(
    grid=(_B // _BB,),
    in_specs=[
        pl.BlockSpec((_BB, _D, _H), lambda i: (i, 0, 0)),
        pl.BlockSpec((_BB, _D, _H), lambda i: (i, 0, 0)),
        pl.BlockSpec((_H, _H), lambda i: (0, 0)),
        pl.BlockSpec((_BB, _D, _D), lambda i: (i, 0, 0)),
        pl.BlockSpec((_BB, _D, _D), lambda i: (i, 0, 0)),
        pl.BlockSpec((_BB, 1, _D), lambda i: (i, 0, 0)),
        pl.BlockSpec((_BB, 1, _D), lambda i: (i, 0, 0)),
        pl.BlockSpec((1, _D), lambda i: (0, 0)),
    ],
    out_specs=[
        pl.BlockSpec((_BB, _D, _D), lambda i: (i, 0, 0)),
        pl.BlockSpec((_BB, _D, _D), lambda i: (i, 0, 0)),
        pl.BlockSpec((_BB, _D), lambda i: (i, 0)),
    ],
    out_shape=[
        jax.ShapeDtypeStruct((_B, _D, _D), jnp.float32),
        jax.ShapeDtypeStruct((_B, _D, _D), jnp.float32),
        jax.ShapeDtypeStruct((_B, _D), jnp.float32),
    ],
)


@jax.jit
def kernel(pos_embed, mage_embed, mask, gumbel, W_policy, dummy_logits):
    mp = jnp.pad(mask[:, :_M].reshape(_B, _D, _D - 1), ((0, 0), (0, 0), (0, 1)))
    gp = jnp.pad(gumbel[:, :_M].reshape(_B, _D, _D - 1), ((0, 0), (0, 0), (0, 1)))
    md = jnp.pad(mask[:, _M:].reshape(_B, 1, _K), ((0, 0), (0, 0), (0, _D - _K)))
    gd = jnp.pad(gumbel[:, _M:].reshape(_B, 1, _K), ((0, 0), (0, 0), (0, _D - _K)))
    dl = jnp.pad(dummy_logits.reshape(1, _K), ((0, 0), (0, _D - _K)))

    samp_p, adj, logp = pl.pallas_call(_body, **_GRID_KWARGS)(
        pos_embed, mage_embed, W_policy, mp, gp, md, gd, dl)

    sample = samp_p[:, :, :_D - 1].reshape(_B, _M)
    return (sample, adj, logp[:, 0])
